# Initial kernel scaffold; baseline (speedup 1.0000x reference)
import jax
import jax.numpy as jnp
from jax.experimental import pallas as pl


def kernel(C3, C4, C5, anchors, params):
    raise NotImplementedError("write your pallas kernel here")


# speedup vs baseline: 2.7067x; 2.7067x over previous
import jax
import jax.numpy as jnp
import numpy as np
from jax import lax
from jax.experimental import pallas as pl
from jax.experimental.pallas import tpu as pltpu

NUM_CLASSES = 80
FEAT = 256
IMG = 1024
MAX_DET = 100
IOU_TH = 0.5
SCORE_TH = 0.05
N_ANCH = 196416
NROW = 1536
NPAD = NROW * 128


def _conv(x, wb, stride=1):
    w, b = wb
    pad = (w.shape[-1] - 1) // 2
    y = lax.conv_general_dilated(x, w, (stride, stride), [(pad, pad), (pad, pad)],
                                 dimension_numbers=('NCHW', 'OIHW', 'NCHW'))
    return y + b[None, :, None, None]


def _up2(x):
    return jnp.repeat(jnp.repeat(x, 2, axis=2), 2, axis=3)


def _head(x, p, prefix, out_dim):
    h = x
    for i in range(1, 5):
        h = jax.nn.relu(_conv(h, p['%s%d' % (prefix, i)]))
    o = _conv(h, p[prefix + '_out'])
    return jnp.transpose(o, (0, 2, 3, 1)).reshape(o.shape[0], -1, out_dim)


def _cls_reduce_kernel(cls_ref, smax_ref, sidx_ref):
    m = cls_ref[0]
    idx = jnp.zeros(m.shape, jnp.int32)
    for c in range(1, NUM_CLASSES):
        v = cls_ref[c]
        gt = v > m
        m = jnp.where(gt, v, m)
        idx = jnp.where(gt, c, idx)
    smax_ref[...] = m
    sidx_ref[...] = idx


def _cls_reduce(cls_t):
    blk = 128
    grid = (NROW // blk,)
    return pl.pallas_call(
        _cls_reduce_kernel,
        grid=grid,
        in_specs=[pl.BlockSpec((NUM_CLASSES, blk, 128), lambda i: (0, i, 0))],
        out_specs=[pl.BlockSpec((blk, 128), lambda i: (i, 0)),
                   pl.BlockSpec((blk, 128), lambda i: (i, 0))],
        out_shape=[jax.ShapeDtypeStruct((NROW, 128), jnp.float32),
                   jax.ShapeDtypeStruct((NROW, 128), jnp.int32)],
        compiler_params=pltpu.CompilerParams(
            dimension_semantics=("parallel",)),
    )(cls_t)


def _nms_kernel(scores_ref, reg_ref, anc_ref,
                oidx_ref, ovalid_ref, ob0_ref, ob1_ref, ob2_ref, ob3_ref,
                s_ref, x1_ref, y1_ref, x2_ref, y2_ref, area_ref):
    a0 = anc_ref[0]
    a1 = anc_ref[1]
    a2 = anc_ref[2]
    a3 = anc_ref[3]
    wa = a2 - a0
    ha = a3 - a1
    cxa = a0 + 0.5 * wa
    cya = a1 + 0.5 * ha
    cx = cxa + reg_ref[0] * 0.1 * wa
    cy = cya + reg_ref[1] * 0.1 * ha
    w = jnp.exp(reg_ref[2] * 0.2) * wa
    h = jnp.exp(reg_ref[3] * 0.2) * ha
    lo = jnp.float32(0.0)
    hi = jnp.float32(IMG)
    x1 = jnp.clip(cx - 0.5 * w, lo, hi)
    y1 = jnp.clip(cy - 0.5 * h, lo, hi)
    x2 = jnp.clip(cx + 0.5 * w, lo, hi)
    y2 = jnp.clip(cy + 0.5 * h, lo, hi)
    x1_ref[...] = x1
    y1_ref[...] = y1
    x2_ref[...] = x2
    y2_ref[...] = y2
    area_ref[...] = (x2 - x1) * (y2 - y1)

    sc = scores_ref[...]
    s_ref[...] = jnp.where(sc > SCORE_TH, sc, -jnp.inf)

    lin = (lax.broadcasted_iota(jnp.int32, (NROW, 128), 0) * 128
           + lax.broadcasted_iota(jnp.int32, (NROW, 128), 1))
    lane = lax.broadcasted_iota(jnp.int32, (1, 128), 1)

    oidx_ref[...] = jnp.zeros((1, 128), jnp.int32)
    ovalid_ref[...] = jnp.zeros((1, 128), jnp.int32)
    ob0_ref[...] = jnp.zeros((1, 128), jnp.float32)
    ob1_ref[...] = jnp.zeros((1, 128), jnp.float32)
    ob2_ref[...] = jnp.zeros((1, 128), jnp.float32)
    ob3_ref[...] = jnp.zeros((1, 128), jnp.float32)

    neg_inf = jnp.float32(-jnp.inf)

    def body(k, carry):
        s = s_ref[...]
        m = jnp.max(s)
        valid = (m > neg_inf).astype(jnp.int32)
        sel = s == m
        idx = jnp.min(jnp.where(sel, lin, jnp.int32(2 ** 30)))
        hit = lin == idx
        xx1v = x1_ref[...]
        yy1v = y1_ref[...]
        xx2v = x2_ref[...]
        yy2v = y2_ref[...]
        b0 = jnp.max(jnp.where(hit, xx1v, neg_inf))
        b1 = jnp.max(jnp.where(hit, yy1v, neg_inf))
        b2 = jnp.max(jnp.where(hit, xx2v, neg_inf))
        b3 = jnp.max(jnp.where(hit, yy2v, neg_inf))
        ia = (b2 - b0) * (b3 - b1)
        xx1 = jnp.maximum(b0, xx1v)
        yy1 = jnp.maximum(b1, yy1v)
        xx2 = jnp.minimum(b2, xx2v)
        yy2 = jnp.minimum(b3, yy2v)
        inter = jnp.maximum(xx2 - xx1, 0.0) * jnp.maximum(yy2 - yy1, 0.0)
        union = jnp.maximum(ia + area_ref[...] - inter, 1e-8)
        kill = (inter > 0.5 * union) | hit
        s_ref[...] = jnp.where(kill, neg_inf, s)
        at_k = lane == k
        oidx_ref[...] = oidx_ref[...] + jnp.where(at_k, idx, 0)
        ovalid_ref[...] = ovalid_ref[...] + jnp.where(at_k, valid, 0)
        ob0_ref[...] = ob0_ref[...] + jnp.where(at_k, b0, 0.0)
        ob1_ref[...] = ob1_ref[...] + jnp.where(at_k, b1, 0.0)
        ob2_ref[...] = ob2_ref[...] + jnp.where(at_k, b2, 0.0)
        ob3_ref[...] = ob3_ref[...] + jnp.where(at_k, b3, 0.0)
        return carry

    lax.fori_loop(0, MAX_DET, body, 0)


def _nms(scores2d, reg_t, anc_t):
    f32 = jnp.float32
    return pl.pallas_call(
        _nms_kernel,
        out_shape=[jax.ShapeDtypeStruct((1, 128), jnp.int32),
                   jax.ShapeDtypeStruct((1, 128), jnp.int32),
                   jax.ShapeDtypeStruct((1, 128), f32),
                   jax.ShapeDtypeStruct((1, 128), f32),
                   jax.ShapeDtypeStruct((1, 128), f32),
                   jax.ShapeDtypeStruct((1, 128), f32)],
        scratch_shapes=[pltpu.VMEM((NROW, 128), f32)] * 6,
        compiler_params=pltpu.CompilerParams(
            vmem_limit_bytes=64 * 1024 * 1024),
    )(scores2d, reg_t, anc_t)


def _to_planes(x, ncol):
    xt = jnp.pad(x.T, ((0, 0), (0, NPAD - N_ANCH)))
    return xt.reshape(ncol, NROW, 128)


def kernel(C3, C4, C5, anchors, params):
    p = params
    P5 = _conv(C5, p['P5_1'])
    P5u = _up2(P5)
    P5 = _conv(P5, p['P5_2'])
    P4 = _up2(_conv(C4, p['P4_1'])) + P5u
    P4u = _up2(P4)
    P4 = _conv(P4, p['P4_2'])
    P3 = _up2(_conv(C3, p['P3_1'])) + P4u
    P3 = _conv(P3, p['P3_2'])
    P6 = _conv(C5, p['P6'], stride=2)
    P7 = _conv(jax.nn.relu(P6), p['P7_2'], stride=2)
    feats = [P3, P4, P5, P6, P7]

    reg = jnp.concatenate([_head(f, p, 'reg', 4) for f in feats], axis=1)[0]
    cls = jax.nn.sigmoid(
        jnp.concatenate([_head(f, p, 'cls', NUM_CLASSES) for f in feats],
                        axis=1))[0]

    cls_t = _to_planes(cls, NUM_CLASSES)
    smax2d, sidx2d = _cls_reduce(cls_t)

    reg_t = _to_planes(reg, 4)
    anc_t = _to_planes(anchors, 4)
    oidx, ovalid, b0, b1, b2, b3 = _nms(smax2d, reg_t, anc_t)

    idxs = oidx[0, :MAX_DET]
    valid = ovalid[0, :MAX_DET] > 0
    vf = valid.astype(jnp.float32)
    flat_s = smax2d.reshape(-1)
    flat_c = sidx2d.reshape(-1)
    nms_scores = flat_s[idxs] * vf
    nms_class = jnp.where(valid, flat_c[idxs], -1)
    nms_boxes = jnp.stack([b0[0, :MAX_DET], b1[0, :MAX_DET],
                           b2[0, :MAX_DET], b3[0, :MAX_DET]],
                          axis=1) * vf[:, None]
    return nms_scores, nms_class, nms_boxes, valid
